# baseline (speedup 1.0000x reference)
import functools

import jax
import jax.numpy as jnp
from jax import lax
from jax.experimental import pallas as pl
from jax.experimental.pallas import tpu as pltpu
from jax.experimental.pallas import tpu_sc as plsc

N = 10000
NPAD = 10240
NC = 2
NS = 16
NW = NC * NS
C = 96
ROWS_PER_TILE = NPAD // NS
ZB = 64
ZCH = ROWS_PER_TILE // ZB

_f32 = jnp.float32


def _make_agg(d, k_chunks):
  mesh = plsc.VectorSubcoreMesh(
      core_axis_name="c", subcore_axis_name="s", num_cores=NC)

  @functools.partial(
      pl.kernel,
      out_type=jax.ShapeDtypeStruct((NC, NPAD, d), _f32),
      mesh=mesh,
      compiler_params=pltpu.CompilerParams(use_tc_tiling_on_sc=False),
      scratch_types=[
          pltpu.VMEM((k_chunks, C), jnp.int32),
          pltpu.VMEM((k_chunks, C), jnp.int32),
          pltpu.VMEM((2, C, d), _f32),
          pltpu.VMEM_SHARED((NPAD, d), _f32),
          pltpu.SemaphoreType.DMA((2,)),
          pltpu.SemaphoreType.DMA((2,)),
      ],
  )
  def agg(table_hbm, srcw_hbm, dstw_hbm, zeros_hbm,
          out_hbm, src_v, dst_v, rows_v, acc_sh, gsem, ssem):
    c = lax.axis_index("c")
    s = lax.axis_index("s")
    wid = s * NC + c
    pltpu.sync_copy(srcw_hbm.at[wid], src_v)
    pltpu.sync_copy(dstw_hbm.at[wid], dst_v)
    zbuf = rows_v.at[0, pl.ds(0, ZB)]
    pltpu.sync_copy(zeros_hbm, zbuf)
    base = s * ROWS_PER_TILE
    for i in range(ZCH):
      pltpu.sync_copy(zbuf, acc_sh.at[pl.ds(base + i * ZB, ZB)])
    plsc.subcore_barrier()

    def gather(k, b):
      return pltpu.make_async_copy(
          table_hbm.at[src_v.at[k]], rows_v.at[b], gsem.at[b])

    def scatter_wait(k, b):
      pltpu.make_async_copy(
          rows_v.at[b], acc_sh.at[dst_v.at[k]], ssem.at[b]).wait()

    def scatter_start(k, b):
      pltpu.async_copy(rows_v.at[b], acc_sh.at[dst_v.at[k]], ssem.at[b],
                       add=True)

    gather(0, 0).start()

    def body(kk, carry):
      k0 = kk * 2

      @pl.when(k0 >= 1)
      def _():
        scatter_wait(k0 - 1, 1)

      gather(k0 + 1, 1).start()
      gather(k0, 0).wait()
      scatter_start(k0, 0)

      scatter_wait(k0, 0)

      @pl.when(k0 + 2 < k_chunks)
      def _():
        gather(k0 + 2, 0).start()

      gather(k0 + 1, 1).wait()
      scatter_start(k0 + 1, 1)
      return carry

    lax.fori_loop(0, k_chunks // 2, body, 0)
    scatter_wait(k_chunks - 1, 1)
    plsc.subcore_barrier()
    for i in range(ZCH):
      pltpu.sync_copy(acc_sh.at[pl.ds(base + i * ZB, ZB)], zbuf)
      pltpu.sync_copy(zbuf, out_hbm.at[c, pl.ds(base + i * ZB, ZB)])

  return agg


def _b1_body(degp_ref, x_ref, w_ref, dinv_ref, hs_ref):
  deg = degp_ref[0, :N, 0:1] + degp_ref[1, :N, 0:1] + 1.0
  dinv = lax.rsqrt(deg)
  h = jnp.dot(x_ref[...], w_ref[...], preferred_element_type=_f32)
  dinv_ref[...] = dinv
  hs_ref[...] = dinv * h


def _mid_body(p_ref, hs_ref, dinv_ref, b_ref, w_ref, out_ref):
  agg = p_ref[0, :N, :] + p_ref[1, :N, :] + hs_ref[...]
  dinv = dinv_ref[...]
  z = jnp.maximum(dinv * agg + b_ref[...], 0.0)
  out_ref[...] = dinv * jnp.dot(z, w_ref[...], preferred_element_type=_f32)


def _fin_body(p_ref, hs_ref, dinv_ref, b_ref, out_ref):
  agg = p_ref[0, :N, :] + p_ref[1, :N, :] + hs_ref[...]
  out_ref[...] = dinv_ref[...] * agg + b_ref[...]


def kernel(x, edge_index, W1, b1, W2, b2, W3, b3):
  e = edge_index.shape[1]
  per_tile = e // NW
  k_chunks = (per_tile + C - 1) // C
  k_chunks += k_chunks % 2
  pad = k_chunks * C - per_tile

  src = jnp.pad(edge_index[0].reshape(NW, per_tile), ((0, 0), (0, pad)))
  dst = jnp.pad(edge_index[1].reshape(NW, per_tile), ((0, 0), (0, pad)),
                constant_values=NPAD - 1)
  srcw = src.reshape(NW, k_chunks, C)
  dstw = dst.reshape(NW, k_chunks, C)

  agg8 = _make_agg(8, k_chunks)
  agg128 = _make_agg(128, k_chunks)
  agg64 = _make_agg(64, k_chunks)
  z8 = jnp.zeros((ZB, 8), _f32)
  z128 = jnp.zeros((ZB, 128), _f32)
  z64 = jnp.zeros((ZB, 64), _f32)

  degp = agg8(jnp.ones((N, 8), _f32), srcw, dstw, z8)

  dinv, hs1 = pl.pallas_call(
      _b1_body,
      out_shape=(jax.ShapeDtypeStruct((N, 1), _f32),
                 jax.ShapeDtypeStruct((N, 128), _f32)),
  )(degp, x, W1)

  p1 = agg128(hs1, srcw, dstw, z128)
  hs2 = pl.pallas_call(
      _mid_body, out_shape=jax.ShapeDtypeStruct((N, 128), _f32),
  )(p1, hs1, dinv, b1.reshape(1, -1), W2)

  p2 = agg128(hs2, srcw, dstw, z128)
  hs3 = pl.pallas_call(
      _mid_body, out_shape=jax.ShapeDtypeStruct((N, 64), _f32),
  )(p2, hs2, dinv, b2.reshape(1, -1), W3)

  p3 = agg64(hs3, srcw, dstw, z64)
  out = pl.pallas_call(
      _fin_body, out_shape=jax.ShapeDtypeStruct((N, 64), _f32),
  )(p3, hs3, dinv, b3.reshape(1, -1))
  return out


# speedup vs baseline: 1.5392x; 1.5392x over previous
import functools

import jax
import jax.numpy as jnp
from jax import lax
from jax.experimental import pallas as pl
from jax.experimental.pallas import tpu as pltpu
from jax.experimental.pallas import tpu_sc as plsc

N = 10000
NPAD = 10240
NC = 2
NS = 16
NW = NC * NS
C = 96
ROWS_PER_TILE = NPAD // NS
ZB = 64
ZCH = ROWS_PER_TILE // ZB

_f32 = jnp.float32


def _make_agg(d, k_chunks):
  mesh = plsc.VectorSubcoreMesh(
      core_axis_name="c", subcore_axis_name="s", num_cores=NC)

  @functools.partial(
      pl.kernel,
      out_type=jax.ShapeDtypeStruct((NC, NPAD, d), _f32),
      mesh=mesh,
      compiler_params=pltpu.CompilerParams(use_tc_tiling_on_sc=False),
      scratch_types=[
          pltpu.VMEM((k_chunks, C), jnp.int32),
          pltpu.VMEM((k_chunks, C), jnp.int32),
          pltpu.VMEM((2, C, d), _f32),
          pltpu.VMEM_SHARED((NPAD, d), _f32),
          pltpu.SemaphoreType.DMA((2,)),
          pltpu.SemaphoreType.DMA((2,)),
      ],
  )
  def agg(table_hbm, srcw_hbm, dstw_hbm, zeros_hbm,
          out_hbm, src_v, dst_v, rows_v, acc_sh, gsem, ssem):
    c = lax.axis_index("c")
    s = lax.axis_index("s")
    wid = s * NC + c
    pltpu.sync_copy(srcw_hbm.at[wid], src_v)
    pltpu.sync_copy(dstw_hbm.at[wid], dst_v)
    zbuf = rows_v.at[0, pl.ds(0, ZB)]
    pltpu.sync_copy(zeros_hbm, zbuf)
    base = s * ROWS_PER_TILE
    for i in range(ZCH):
      pltpu.sync_copy(zbuf, acc_sh.at[pl.ds(base + i * ZB, ZB)])
    plsc.subcore_barrier()

    def gather(k, b):
      return pltpu.make_async_copy(
          table_hbm.at[src_v.at[k]], rows_v.at[b], gsem.at[b])

    def scatter_wait(k, b):
      pltpu.make_async_copy(
          rows_v.at[b], acc_sh.at[dst_v.at[k]], ssem.at[b]).wait()

    gather(0, 0).start()

    def body(k, carry):
      b = lax.rem(k, 2)
      nb = lax.rem(k + 1, 2)

      @pl.when(k >= 1)
      def _():
        scatter_wait(k - 1, nb)

      @pl.when(k + 1 < k_chunks)
      def _():
        gather(k + 1, nb).start()

      gather(k, b).wait()
      pltpu.async_copy(rows_v.at[b], acc_sh.at[dst_v.at[k]], ssem.at[b],
                       add=True)
      return carry

    lax.fori_loop(0, k_chunks, body, 0)
    scatter_wait(k_chunks - 1, lax.rem(k_chunks - 1, 2))
    plsc.subcore_barrier()
    for i in range(ZCH):
      pltpu.sync_copy(acc_sh.at[pl.ds(base + i * ZB, ZB)], zbuf)
      pltpu.sync_copy(zbuf, out_hbm.at[c, pl.ds(base + i * ZB, ZB)])

  return agg


def _make_deg(k_chunks):
  mesh = plsc.VectorSubcoreMesh(
      core_axis_name="c", subcore_axis_name="s", num_cores=NC)

  @functools.partial(
      pl.kernel,
      out_type=jax.ShapeDtypeStruct((NC, NPAD, 8), _f32),
      mesh=mesh,
      compiler_params=pltpu.CompilerParams(use_tc_tiling_on_sc=False),
      scratch_types=[
          pltpu.VMEM((k_chunks, C), jnp.int32),
          pltpu.VMEM((C, 8), _f32),
          pltpu.VMEM((ZB, 8), _f32),
          pltpu.SemaphoreType.DMA((2,)),
          pltpu.VMEM_SHARED((NPAD, 8), _f32),
      ],
  )
  def deg(ones_hbm, dstw_hbm, zeros_hbm, out_hbm,
          dst_v, ones_v, zv, ssem, acc_sh):
    c = lax.axis_index("c")
    s = lax.axis_index("s")
    wid = s * NC + c
    pltpu.sync_copy(dstw_hbm.at[wid], dst_v)
    pltpu.sync_copy(ones_hbm, ones_v)
    pltpu.sync_copy(zeros_hbm, zv)
    base = s * ROWS_PER_TILE
    for i in range(ZCH):
      pltpu.sync_copy(zv, acc_sh.at[pl.ds(base + i * ZB, ZB)])
    plsc.subcore_barrier()

    def body(k, carry):
      b = lax.rem(k, 2)

      @pl.when(k >= 2)
      def _():
        pltpu.make_async_copy(
            ones_v, acc_sh.at[dst_v.at[k - 2]], ssem.at[b]).wait()

      pltpu.async_copy(ones_v, acc_sh.at[dst_v.at[k]], ssem.at[b], add=True)
      return carry

    lax.fori_loop(0, k_chunks, body, 0)
    for k in (k_chunks - 2, k_chunks - 1):
      pltpu.make_async_copy(
          ones_v, acc_sh.at[dst_v.at[k]], ssem.at[lax.rem(k, 2)]).wait()
    plsc.subcore_barrier()
    for i in range(ZCH):
      pltpu.sync_copy(acc_sh.at[pl.ds(base + i * ZB, ZB)], zv)
      pltpu.sync_copy(zv, out_hbm.at[c, pl.ds(base + i * ZB, ZB)])

  return deg


def _bx_body(x_ref, w_ref, h_ref):
  h_ref[...] = jnp.dot(x_ref[...], w_ref[...], preferred_element_type=_f32)


def _b1_body(degp_ref, h_ref, dinv_ref, hs_ref):
  deg = degp_ref[0, :N, 0:1] + degp_ref[1, :N, 0:1] + 1.0
  dinv = lax.rsqrt(deg)
  dinv_ref[...] = dinv
  hs_ref[...] = dinv * h_ref[...]


def _mid_body(p_ref, hs_ref, dinv_ref, b_ref, w_ref, out_ref):
  agg = p_ref[0, :N, :] + p_ref[1, :N, :] + hs_ref[...]
  dinv = dinv_ref[...]
  z = jnp.maximum(dinv * agg + b_ref[...], 0.0)
  out_ref[...] = dinv * jnp.dot(z, w_ref[...], preferred_element_type=_f32)


def _fin_body(p_ref, hs_ref, dinv_ref, b_ref, out_ref):
  agg = p_ref[0, :N, :] + p_ref[1, :N, :] + hs_ref[...]
  out_ref[...] = dinv_ref[...] * agg + b_ref[...]


def kernel(x, edge_index, W1, b1, W2, b2, W3, b3):
  e = edge_index.shape[1]
  per_tile = e // NW
  k_chunks = (per_tile + C - 1) // C
  pad = k_chunks * C - per_tile

  src = jnp.pad(edge_index[0].reshape(NW, per_tile), ((0, 0), (0, pad)))
  dst = jnp.pad(edge_index[1].reshape(NW, per_tile), ((0, 0), (0, pad)),
                constant_values=NPAD - 1)
  srcw = src.reshape(NW, k_chunks, C)
  dstw = dst.reshape(NW, k_chunks, C)

  degk = _make_deg(k_chunks)
  agg128 = _make_agg(128, k_chunks)
  agg64 = _make_agg(64, k_chunks)
  z8 = jnp.zeros((ZB, 8), _f32)
  z128 = jnp.zeros((ZB, 128), _f32)
  z64 = jnp.zeros((ZB, 64), _f32)

  degp = degk(jnp.ones((C, 8), _f32), dstw, z8)
  h1 = pl.pallas_call(
      _bx_body, out_shape=jax.ShapeDtypeStruct((N, 128), _f32),
  )(x, W1)

  dinv, hs1 = pl.pallas_call(
      _b1_body,
      out_shape=(jax.ShapeDtypeStruct((N, 1), _f32),
                 jax.ShapeDtypeStruct((N, 128), _f32)),
  )(degp, h1)

  p1 = agg128(hs1, srcw, dstw, z128)
  hs2 = pl.pallas_call(
      _mid_body, out_shape=jax.ShapeDtypeStruct((N, 128), _f32),
  )(p1, hs1, dinv, b1.reshape(1, -1), W2)

  p2 = agg128(hs2, srcw, dstw, z128)
  hs3 = pl.pallas_call(
      _mid_body, out_shape=jax.ShapeDtypeStruct((N, 64), _f32),
  )(p2, hs2, dinv, b2.reshape(1, -1), W3)

  p3 = agg64(hs3, srcw, dstw, z64)
  out = pl.pallas_call(
      _fin_body, out_shape=jax.ShapeDtypeStruct((N, 64), _f32),
  )(p3, hs3, dinv, b3.reshape(1, -1))
  return out
